# Initial kernel scaffold; baseline (speedup 1.0000x reference)
import functools

import jax
import jax.numpy as jnp
from jax import lax
from jax.experimental import pallas as pl
from jax.experimental.pallas import tpu as pltpu
from jax.experimental.pallas import tpu_sc as plsc

NW = 32

C1 = 512
C2 = 1024


def _make_embed_kernel(n_total, d_word, d_tag, d_rel):
    per_w = n_total // NW
    mesh = plsc.VectorSubcoreMesh(core_axis_name="c", subcore_axis_name="s")

    @functools.partial(
        pl.kernel,
        mesh=mesh,
        out_type=(
            jax.ShapeDtypeStruct((n_total, d_word), jnp.float32),
            jax.ShapeDtypeStruct((n_total, d_tag), jnp.float32),
            jax.ShapeDtypeStruct((n_total, d_rel), jnp.float32),
        ),
        scratch_types=[
            pltpu.VMEM((C1,), jnp.int32),
            pltpu.VMEM((C1, d_word), jnp.float32),
            pltpu.VMEM((C2,), jnp.int32),
            pltpu.VMEM((C2, d_tag), jnp.float32),
            pltpu.SemaphoreType.DMA,
        ],
    )
    def embed_kernel(sent_idx, tag_idx, rel_idx, w_word, w_tag, w_rel,
                     out_sent, out_tag, out_rel,
                     idx1_v, rows1_v, idx2_v, rows2_v, sem):
        wid = lax.axis_index("s") * 2 + lax.axis_index("c")
        base = wid * per_w

        def do_phase(idx_hbm, table_hbm, out_hbm, idx_v, rows_v, chunk):
            def body(g, carry):
                off = base + g * chunk
                pltpu.sync_copy(idx_hbm.at[pl.ds(off, chunk)], idx_v)
                pltpu.async_copy(table_hbm.at[idx_v], rows_v, sem).wait()
                pltpu.sync_copy(rows_v, out_hbm.at[pl.ds(off, chunk)])
                return carry
            lax.fori_loop(0, per_w // chunk, body, 0)

        do_phase(sent_idx, w_word, out_sent, idx1_v, rows1_v, C1)
        do_phase(tag_idx, w_tag, out_tag, idx2_v, rows2_v, C2)
        do_phase(rel_idx, w_rel, out_rel, idx2_v, rows2_v, C2)

    return embed_kernel


def kernel(sent_inputs, tag_inputs, rel_inputs, W_word, W_tag, W_rel):
    b, s = sent_inputs.shape
    n = b * s
    d_word = W_word.shape[1]
    d_tag = W_tag.shape[1]
    d_rel = W_rel.shape[1]
    sent_flat = sent_inputs.reshape(-1).astype(jnp.int32)
    tag_flat = tag_inputs.reshape(-1).astype(jnp.int32)
    rel_flat = rel_inputs.reshape(-1).astype(jnp.int32)
    fn = _make_embed_kernel(n, d_word, d_tag, d_rel)
    o_sent, o_tag, o_rel = fn(sent_flat, tag_flat, rel_flat, W_word, W_tag, W_rel)
    return (
        o_sent.reshape(b, 1, s, d_word),
        o_tag.reshape(b, 1, s, d_tag),
        o_rel.reshape(b, 1, s, d_rel),
    )


# speedup vs baseline: 1.9531x; 1.9531x over previous
import functools

import jax
import jax.numpy as jnp
from jax import lax
from jax.experimental import pallas as pl
from jax.experimental.pallas import tpu as pltpu
from jax.experimental.pallas import tpu_sc as plsc

NW = 32

C1 = 512
C2 = 1024


def _make_embed_kernel(n_total, d_word, d_tag, d_rel):
    per_w = n_total // NW
    mesh = plsc.VectorSubcoreMesh(core_axis_name="c", subcore_axis_name="s")

    @functools.partial(
        pl.kernel,
        mesh=mesh,
        compiler_params=pltpu.CompilerParams(use_tc_tiling_on_sc=False),
        out_type=(
            jax.ShapeDtypeStruct((n_total, d_word), jnp.float32),
            jax.ShapeDtypeStruct((n_total, d_tag), jnp.float32),
            jax.ShapeDtypeStruct((n_total, d_rel), jnp.float32),
        ),
        scratch_types=[
            pltpu.VMEM((C1,), jnp.int32),
            pltpu.VMEM((C1, d_word), jnp.float32),
            pltpu.VMEM((C2,), jnp.int32),
            pltpu.VMEM((C2, d_tag), jnp.float32),
            pltpu.SemaphoreType.DMA,
        ],
    )
    def embed_kernel(sent_idx, tag_idx, rel_idx, w_word, w_tag, w_rel,
                     out_sent, out_tag, out_rel,
                     idx1_v, rows1_v, idx2_v, rows2_v, sem):
        wid = lax.axis_index("s") * 2 + lax.axis_index("c")
        base = wid * per_w

        def do_phase(idx_hbm, table_hbm, out_hbm, idx_v, rows_v, chunk):
            def body(g, carry):
                off = base + g * chunk
                pltpu.sync_copy(idx_hbm.at[pl.ds(off, chunk)], idx_v)
                pltpu.async_copy(table_hbm.at[idx_v], rows_v, sem).wait()
                pltpu.sync_copy(rows_v, out_hbm.at[pl.ds(off, chunk)])
                return carry
            lax.fori_loop(0, per_w // chunk, body, 0)

        do_phase(sent_idx, w_word, out_sent, idx1_v, rows1_v, C1)
        do_phase(tag_idx, w_tag, out_tag, idx2_v, rows2_v, C2)
        do_phase(rel_idx, w_rel, out_rel, idx2_v, rows2_v, C2)

    return embed_kernel


def kernel(sent_inputs, tag_inputs, rel_inputs, W_word, W_tag, W_rel):
    b, s = sent_inputs.shape
    n = b * s
    d_word = W_word.shape[1]
    d_tag = W_tag.shape[1]
    d_rel = W_rel.shape[1]
    sent_flat = sent_inputs.reshape(-1).astype(jnp.int32)
    tag_flat = tag_inputs.reshape(-1).astype(jnp.int32)
    rel_flat = rel_inputs.reshape(-1).astype(jnp.int32)
    fn = _make_embed_kernel(n, d_word, d_tag, d_rel)
    o_sent, o_tag, o_rel = fn(sent_flat, tag_flat, rel_flat, W_word, W_tag, W_rel)
    return (
        o_sent.reshape(b, 1, s, d_word),
        o_tag.reshape(b, 1, s, d_tag),
        o_rel.reshape(b, 1, s, d_rel),
    )


# speedup vs baseline: 1.9839x; 1.0157x over previous
import functools

import jax
import jax.numpy as jnp
from jax import lax
from jax.experimental import pallas as pl
from jax.experimental.pallas import tpu as pltpu
from jax.experimental.pallas import tpu_sc as plsc

NW = 32
NB = 2
C1 = 512
C2 = 512


def _make_embed_kernel(n_total, d_word, d_tag, d_rel):
    per_w = n_total // NW
    mesh = plsc.VectorSubcoreMesh(core_axis_name="c", subcore_axis_name="s")

    @functools.partial(
        pl.kernel,
        mesh=mesh,
        compiler_params=pltpu.CompilerParams(use_tc_tiling_on_sc=False),
        out_type=(
            jax.ShapeDtypeStruct((n_total, d_word), jnp.float32),
            jax.ShapeDtypeStruct((n_total, d_tag), jnp.float32),
            jax.ShapeDtypeStruct((n_total, d_rel), jnp.float32),
        ),
        scratch_types=[
            pltpu.VMEM((per_w,), jnp.int32),
            pltpu.VMEM((NB, C1, d_word), jnp.float32),
            pltpu.VMEM((NB, C2, d_tag), jnp.float32),
            pltpu.SemaphoreType.DMA,
            pltpu.SemaphoreType.DMA,
            pltpu.SemaphoreType.DMA,
            pltpu.SemaphoreType.DMA,
        ],
    )
    def embed_kernel(sent_idx, tag_idx, rel_idx, w_word, w_tag, w_rel,
                     out_sent, out_tag, out_rel,
                     idx_all, rows1, rows2, sg0, sg1, so0, so1):
        wid = lax.axis_index("s") * 2 + lax.axis_index("c")
        base = wid * per_w
        sem_g = (sg0, sg1)
        sem_o = (so0, so1)

        def do_phase(idx_hbm, table_hbm, out_hbm, rows, chunk):
            n_outer = per_w // (chunk * NB)
            pltpu.sync_copy(idx_hbm.at[pl.ds(base, per_w)], idx_all)

            def wait_out(b):
                pltpu.make_async_copy(
                    rows.at[b], out_hbm.at[pl.ds(base, chunk)], sem_o[b]
                ).wait()

            def outer(o, carry):
                gathers = []
                for b in range(NB):
                    g = o * NB + b

                    @pl.when(o >= 1)
                    def _():
                        wait_out(b)

                    gathers.append(pltpu.async_copy(
                        table_hbm.at[idx_all.at[pl.ds(g * chunk, chunk)]],
                        rows.at[b], sem_g[b]))
                for b in range(NB):
                    g = o * NB + b
                    gathers[b].wait()
                    pltpu.async_copy(
                        rows.at[b], out_hbm.at[pl.ds(base + g * chunk, chunk)],
                        sem_o[b])
                return carry

            lax.fori_loop(0, n_outer, outer, 0)
            for b in range(NB):
                wait_out(b)

        do_phase(sent_idx, w_word, out_sent, rows1, C1)
        do_phase(tag_idx, w_tag, out_tag, rows2, C2)
        do_phase(rel_idx, w_rel, out_rel, rows2, C2)

    return embed_kernel


def kernel(sent_inputs, tag_inputs, rel_inputs, W_word, W_tag, W_rel):
    b, s = sent_inputs.shape
    n = b * s
    d_word = W_word.shape[1]
    d_tag = W_tag.shape[1]
    d_rel = W_rel.shape[1]
    sent_flat = sent_inputs.reshape(-1).astype(jnp.int32)
    tag_flat = tag_inputs.reshape(-1).astype(jnp.int32)
    rel_flat = rel_inputs.reshape(-1).astype(jnp.int32)
    fn = _make_embed_kernel(n, d_word, d_tag, d_rel)
    o_sent, o_tag, o_rel = fn(sent_flat, tag_flat, rel_flat, W_word, W_tag, W_rel)
    return (
        o_sent.reshape(b, 1, s, d_word),
        o_tag.reshape(b, 1, s, d_tag),
        o_rel.reshape(b, 1, s, d_rel),
    )


# speedup vs baseline: 2.7085x; 1.3653x over previous
import functools

import jax
import jax.numpy as jnp
from jax import lax
from jax.experimental import pallas as pl
from jax.experimental.pallas import tpu as pltpu
from jax.experimental.pallas import tpu_sc as plsc

NW = 32
LANES = 16


def _make_embed_kernel(bsz, seq, d_word, d_tag, d_rel, n_packed):
    assert bsz == 128 * NW and seq % 2 == 0
    assert d_word == 64 and d_tag == 32 and d_rel == 32
    mesh = plsc.VectorSubcoreMesh(core_axis_name="c", subcore_axis_name="s")

    @functools.partial(
        pl.kernel,
        mesh=mesh,
        compiler_params=pltpu.CompilerParams(needs_layout_passes=False),
        out_type=(
            jax.ShapeDtypeStruct((seq, d_word, bsz), jnp.float32),
            jax.ShapeDtypeStruct((seq, d_tag, bsz), jnp.float32),
            jax.ShapeDtypeStruct((seq, d_rel, bsz), jnp.float32),
        ),
        scratch_types=[
            pltpu.VMEM((seq, 128), jnp.int32),
            pltpu.VMEM((2, 128), jnp.int32),
            pltpu.VMEM((2, 128, 128), jnp.float32),
            pltpu.VMEM((2, d_word, 128), jnp.float32),
            pltpu.VMEM((d_tag, 64), jnp.float32),
            pltpu.VMEM((2, d_tag, 128), jnp.float32),
            pltpu.SemaphoreType.DMA,
            pltpu.SemaphoreType.DMA,
            pltpu.SemaphoreType.DMA,
            pltpu.SemaphoreType.DMA,
        ],
    )
    def embed_kernel(sent_t, tag_t, rel_t, w2, wtag_t, wrel_t,
                     o_word, o_tag, o_rel,
                     idx_v, pidx, rows, out_w, wtab, out_s,
                     sg0, sg1, so0, so1):
        wid = lax.axis_index("s") * 2 + lax.axis_index("c")
        bcol = wid * 128
        sem_g = (sg0, sg1)
        sem_o = (so0, so1)
        ii = lax.iota(jnp.int32, LANES)

        def prep_and_fire_gather(s, b):
            pidx_b = pidx.at[b]
            for jc in range(8):
                idx16 = idx_v[s, pl.ds(jc * LANES, LANES)]
                pidx_b[pl.ds(jc * LANES, LANES)] = lax.shift_right_logical(idx16, 1)
            return pltpu.async_copy(w2.at[pidx.at[b]], rows.at[b], sem_g[b])

        def wait_gather(b):
            pltpu.make_async_copy(w2.at[pidx.at[b]], rows.at[b], sem_g[b]).wait()

        def transpose_word(s, b):
            rows_b = rows.at[b]
            out_b = out_w.at[b]

            def jbody(jc, carry):
                j0 = jc * LANES
                idx16 = idx_v[s, pl.ds(j0, LANES)]
                col_base = (idx16 & 1) * d_word
                rvec = j0 + ii
                for d in range(d_word):
                    g = plsc.load_gather(rows_b, [rvec, col_base + d])
                    out_b[d, pl.ds(j0, LANES)] = g
                return carry

            lax.fori_loop(0, 8, jbody, 0)

        def wait_out_word(b, s):
            pltpu.make_async_copy(
                out_w.at[b], o_word.at[s, :, pl.ds(bcol, 128)], sem_o[b]).wait()

        pltpu.sync_copy(sent_t.at[:, pl.ds(bcol, 128)], idx_v)
        c0 = prep_and_fire_gather(0, 0)
        del c0

        def word_outer(o, carry):
            for b in range(2):
                s = 2 * o + b
                if b == 0:
                    prep_and_fire_gather(s + 1, 1)
                else:
                    @pl.when(o < seq // 2 - 1)
                    def _():
                        prep_and_fire_gather(s + 1, 0)
                wait_gather(b)

                @pl.when(o >= 1)
                def _():
                    wait_out_word(b, s)

                transpose_word(s, b)
                pltpu.async_copy(
                    out_w.at[b], o_word.at[s, :, pl.ds(bcol, 128)], sem_o[b])
            return carry

        lax.fori_loop(0, seq // 2, word_outer, 0)
        for b in range(2):
            wait_out_word(b, 0)

        def small_phase(idx_hbm, table_hbm, out_hbm, d_out):
            pltpu.sync_copy(idx_hbm.at[:, pl.ds(bcol, 128)], idx_v)
            pltpu.sync_copy(table_hbm, wtab)

            def wait_out(b, s):
                pltpu.make_async_copy(
                    out_s.at[b], out_hbm.at[s, :, pl.ds(bcol, 128)], sem_o[b]).wait()

            def outer(o, carry):
                for b in range(2):
                    s = 2 * o + b

                    @pl.when(o >= 1)
                    def _():
                        wait_out(b, s)

                    out_b = out_s.at[b]

                    def jbody(jc, carry2):
                        j0 = jc * LANES
                        idx16 = idx_v[s, pl.ds(j0, LANES)]
                        for d in range(d_out):
                            g = plsc.load_gather(
                                wtab, [jnp.full((LANES,), d, jnp.int32), idx16])
                            out_b[d, pl.ds(j0, LANES)] = g
                        return carry2

                    lax.fori_loop(0, 8, jbody, 0)
                    pltpu.async_copy(
                        out_s.at[b], out_hbm.at[s, :, pl.ds(bcol, 128)], sem_o[b])
                return carry

            lax.fori_loop(0, seq // 2, outer, 0)
            for b in range(2):
                wait_out(b, 0)

        small_phase(tag_t, wtag_t, o_tag, d_tag)
        small_phase(rel_t, wrel_t, o_rel, d_rel)

    return embed_kernel


def kernel(sent_inputs, tag_inputs, rel_inputs, W_word, W_tag, W_rel):
    bsz, seq = sent_inputs.shape
    n_vocab, d_word = W_word.shape
    d_tag = W_tag.shape[1]
    d_rel = W_rel.shape[1]
    pack = 128 // d_word
    w2 = W_word.reshape(n_vocab // pack, 128)
    fn = _make_embed_kernel(bsz, seq, d_word, d_tag, d_rel, n_vocab // pack)
    o_w, o_t, o_r = fn(
        sent_inputs.T.astype(jnp.int32),
        tag_inputs.T.astype(jnp.int32),
        rel_inputs.T.astype(jnp.int32),
        w2, W_tag.T, W_rel.T)
    return (
        jnp.expand_dims(jnp.transpose(o_w, (2, 0, 1)), 1),
        jnp.expand_dims(jnp.transpose(o_t, (2, 0, 1)), 1),
        jnp.expand_dims(jnp.transpose(o_r, (2, 0, 1)), 1),
    )


# speedup vs baseline: 4.0511x; 1.4957x over previous
import functools

import jax
import jax.numpy as jnp
from jax import lax
from jax.experimental import pallas as pl
from jax.experimental.pallas import tpu as pltpu
from jax.experimental.pallas import tpu_sc as plsc

NW = 32
LANES = 16


def _make_embed_kernel(bsz, seq, d_word, d_tag, d_rel, n_packed):
    assert bsz == 128 * NW and seq % 2 == 0
    assert d_word == 64 and d_tag == 32 and d_rel == 32
    mesh = plsc.VectorSubcoreMesh(core_axis_name="c", subcore_axis_name="s")

    @functools.partial(
        pl.kernel,
        mesh=mesh,
        compiler_params=pltpu.CompilerParams(needs_layout_passes=False),
        out_type=(
            jax.ShapeDtypeStruct((seq, d_word, bsz), jnp.float32),
            jax.ShapeDtypeStruct((seq, d_tag, bsz), jnp.float32),
            jax.ShapeDtypeStruct((seq, d_rel, bsz), jnp.float32),
        ),
        scratch_types=[
            pltpu.VMEM((seq, 128), jnp.int32),
            pltpu.VMEM((2, 128), jnp.int32),
            pltpu.VMEM((2, 128, 128), jnp.float32),
            pltpu.VMEM((2, d_word, 128), jnp.float32),
            pltpu.VMEM((d_tag, 64), jnp.float32),
            pltpu.VMEM((2, d_tag, 128), jnp.float32),
            pltpu.SemaphoreType.DMA,
            pltpu.SemaphoreType.DMA,
            pltpu.SemaphoreType.DMA,
            pltpu.SemaphoreType.DMA,
        ],
    )
    def embed_kernel(sent_t, tag_t, rel_t, w2, wtag_t, wrel_t,
                     o_word, o_tag, o_rel,
                     idx_v, pidx, rows, out_w, wtab, out_s,
                     sg0, sg1, so0, so1):
        wid = lax.axis_index("s") * 2 + lax.axis_index("c")
        bcol = wid * 128
        sem_g = (sg0, sg1)
        sem_o = (so0, so1)
        ii = lax.iota(jnp.int32, LANES)

        def prep_and_fire_gather(s, b):
            pidx_b = pidx.at[b]
            for jc in range(8):
                idx16 = idx_v[s, pl.ds(jc * LANES, LANES)]
                pidx_b[pl.ds(jc * LANES, LANES)] = lax.shift_right_logical(idx16, 1)
            return pltpu.async_copy(w2.at[pidx.at[b]], rows.at[b], sem_g[b])

        def wait_gather(b):
            pltpu.make_async_copy(w2.at[pidx.at[b]], rows.at[b], sem_g[b]).wait()

        def transpose_word(s, b):
            rows_b = rows.at[b]
            out_b = out_w.at[b]

            def jbody(jc, carry):
                j0 = jc * LANES
                idx16 = idx_v[s, pl.ds(j0, LANES)]
                col_base = (idx16 & 1) * d_word
                rvec = j0 + ii
                for d0 in range(0, d_word, 8):
                    gs = [plsc.load_gather(rows_b, [rvec, col_base + (d0 + k)])
                          for k in range(8)]
                    for k in range(8):
                        out_b[d0 + k, pl.ds(j0, LANES)] = gs[k]
                return carry

            lax.fori_loop(0, 8, jbody, 0)

        def wait_out_word(b, s):
            pltpu.make_async_copy(
                out_w.at[b], o_word.at[s, :, pl.ds(bcol, 128)], sem_o[b]).wait()

        pltpu.sync_copy(sent_t.at[:, pl.ds(bcol, 128)], idx_v)
        c0 = prep_and_fire_gather(0, 0)
        del c0

        def word_outer(o, carry):
            for b in range(2):
                s = 2 * o + b
                if b == 0:
                    prep_and_fire_gather(s + 1, 1)
                else:
                    @pl.when(o < seq // 2 - 1)
                    def _():
                        prep_and_fire_gather(s + 1, 0)
                wait_gather(b)

                @pl.when(o >= 1)
                def _():
                    wait_out_word(b, s)

                transpose_word(s, b)
                pltpu.async_copy(
                    out_w.at[b], o_word.at[s, :, pl.ds(bcol, 128)], sem_o[b])
            return carry

        lax.fori_loop(0, seq // 2, word_outer, 0)
        for b in range(2):
            wait_out_word(b, 0)

        def small_phase(idx_hbm, table_hbm, out_hbm, d_out):
            pltpu.sync_copy(idx_hbm.at[:, pl.ds(bcol, 128)], idx_v)
            pltpu.sync_copy(table_hbm, wtab)

            def wait_out(b, s):
                pltpu.make_async_copy(
                    out_s.at[b], out_hbm.at[s, :, pl.ds(bcol, 128)], sem_o[b]).wait()

            def outer(o, carry):
                for b in range(2):
                    s = 2 * o + b

                    @pl.when(o >= 1)
                    def _():
                        wait_out(b, s)

                    out_b = out_s.at[b]

                    def jbody(jc, carry2):
                        j0 = jc * LANES
                        idx16 = idx_v[s, pl.ds(j0, LANES)]
                        for d0 in range(0, d_out, 8):
                            gs = [plsc.load_gather(
                                wtab,
                                [jnp.full((LANES,), d0 + k, jnp.int32), idx16])
                                for k in range(8)]
                            for k in range(8):
                                out_b[d0 + k, pl.ds(j0, LANES)] = gs[k]
                        return carry2

                    lax.fori_loop(0, 8, jbody, 0)
                    pltpu.async_copy(
                        out_s.at[b], out_hbm.at[s, :, pl.ds(bcol, 128)], sem_o[b])
                return carry

            lax.fori_loop(0, seq // 2, outer, 0)
            for b in range(2):
                wait_out(b, 0)

        small_phase(tag_t, wtag_t, o_tag, d_tag)
        small_phase(rel_t, wrel_t, o_rel, d_rel)

    return embed_kernel


def kernel(sent_inputs, tag_inputs, rel_inputs, W_word, W_tag, W_rel):
    bsz, seq = sent_inputs.shape
    n_vocab, d_word = W_word.shape
    d_tag = W_tag.shape[1]
    d_rel = W_rel.shape[1]
    pack = 128 // d_word
    w2 = W_word.reshape(n_vocab // pack, 128)
    fn = _make_embed_kernel(bsz, seq, d_word, d_tag, d_rel, n_vocab // pack)
    o_w, o_t, o_r = fn(
        sent_inputs.T.astype(jnp.int32),
        tag_inputs.T.astype(jnp.int32),
        rel_inputs.T.astype(jnp.int32),
        w2, W_tag.T, W_rel.T)
    return (
        jnp.expand_dims(jnp.transpose(o_w, (2, 0, 1)), 1),
        jnp.expand_dims(jnp.transpose(o_t, (2, 0, 1)), 1),
        jnp.expand_dims(jnp.transpose(o_r, (2, 0, 1)), 1),
    )


# speedup vs baseline: 4.4630x; 1.1017x over previous
import functools

import jax
import jax.numpy as jnp
from jax import lax
from jax.experimental import pallas as pl
from jax.experimental.pallas import tpu as pltpu
from jax.experimental.pallas import tpu_sc as plsc

NW = 32
LANES = 16


def _make_word_kernel(bsz, seq, d_word):
    assert bsz == 128 * NW and seq % 2 == 0 and d_word == 64
    mesh = plsc.VectorSubcoreMesh(core_axis_name="c", subcore_axis_name="s")

    @functools.partial(
        pl.kernel,
        mesh=mesh,
        compiler_params=pltpu.CompilerParams(needs_layout_passes=False),
        out_type=jax.ShapeDtypeStruct((seq, d_word, bsz), jnp.float32),
        scratch_types=[
            pltpu.VMEM((seq, 128), jnp.int32),
            pltpu.VMEM((2, 128), jnp.int32),
            pltpu.VMEM((2, 128, 128), jnp.float32),
            pltpu.VMEM((2, d_word, 128), jnp.float32),
            pltpu.SemaphoreType.DMA,
            pltpu.SemaphoreType.DMA,
            pltpu.SemaphoreType.DMA,
            pltpu.SemaphoreType.DMA,
        ],
    )
    def word_kernel(sent_t, w2, o_word, idx_v, pidx, rows, out_w,
                    sg0, sg1, so0, so1):
        wid = lax.axis_index("s") * 2 + lax.axis_index("c")
        bcol = wid * 128
        sem_g = (sg0, sg1)
        sem_o = (so0, so1)
        ii = lax.iota(jnp.int32, LANES)

        def prep_and_fire_gather(s, b):
            pidx_b = pidx.at[b]
            for jc in range(8):
                idx16 = idx_v[s, pl.ds(jc * LANES, LANES)]
                pidx_b[pl.ds(jc * LANES, LANES)] = lax.shift_right_logical(idx16, 1)
            pltpu.async_copy(w2.at[pidx.at[b]], rows.at[b], sem_g[b])

        def wait_gather(b):
            pltpu.make_async_copy(w2.at[pidx.at[b]], rows.at[b], sem_g[b]).wait()

        def transpose_word(s, b):
            rows_b = rows.at[b]
            out_b = out_w.at[b]

            def jbody(jc, carry):
                j0 = jc * LANES
                idx16 = idx_v[s, pl.ds(j0, LANES)]
                col_base = (idx16 & 1) * d_word
                rvec = j0 + ii
                for d0 in range(0, d_word, 8):
                    gs = [plsc.load_gather(rows_b, [rvec, col_base + (d0 + k)])
                          for k in range(8)]
                    for k in range(8):
                        out_b[d0 + k, pl.ds(j0, LANES)] = gs[k]
                return carry

            lax.fori_loop(0, 8, jbody, 0)

        def wait_out(b, s):
            pltpu.make_async_copy(
                out_w.at[b], o_word.at[s, :, pl.ds(bcol, 128)], sem_o[b]).wait()

        pltpu.sync_copy(sent_t.at[:, pl.ds(bcol, 128)], idx_v)
        prep_and_fire_gather(0, 0)

        def outer(o, carry):
            for b in range(2):
                s = 2 * o + b
                if b == 0:
                    prep_and_fire_gather(s + 1, 1)
                else:
                    @pl.when(o < seq // 2 - 1)
                    def _():
                        prep_and_fire_gather(s + 1, 0)
                wait_gather(b)

                @pl.when(o >= 1)
                def _():
                    wait_out(b, s)

                transpose_word(s, b)
                pltpu.async_copy(
                    out_w.at[b], o_word.at[s, :, pl.ds(bcol, 128)], sem_o[b])
            return carry

        lax.fori_loop(0, seq // 2, outer, 0)
        for b in range(2):
            wait_out(b, 0)

    return word_kernel


def _make_small_kernel(bsz, seq, d_tag, d_rel):
    assert bsz == 128 * NW and seq % 2 == 0
    assert d_tag % 8 == 0 and d_rel % 8 == 0
    mesh = plsc.VectorSubcoreMesh(core_axis_name="c", subcore_axis_name="s")

    @functools.partial(
        pl.kernel,
        mesh=mesh,
        compiler_params=pltpu.CompilerParams(needs_layout_passes=False),
        out_type=(
            jax.ShapeDtypeStruct((seq, d_tag, bsz), jnp.float32),
            jax.ShapeDtypeStruct((seq, d_rel, bsz), jnp.float32),
        ),
        scratch_types=[
            pltpu.VMEM((seq, 128), jnp.int32),
            pltpu.VMEM((d_tag, 64), jnp.float32),
            pltpu.VMEM((2, d_tag, 128), jnp.float32),
            pltpu.SemaphoreType.DMA,
            pltpu.SemaphoreType.DMA,
        ],
    )
    def small_kernel(tag_t, rel_t, wtag_t, wrel_t, o_tag, o_rel,
                     idx_v, wtab, out_s, so0, so1):
        wid = lax.axis_index("s") * 2 + lax.axis_index("c")
        bcol = wid * 128
        sem_o = (so0, so1)

        def phase(idx_hbm, table_hbm, out_hbm, d_out):
            pltpu.sync_copy(idx_hbm.at[:, pl.ds(bcol, 128)], idx_v)
            pltpu.sync_copy(table_hbm, wtab)

            def wait_out(b, s):
                pltpu.make_async_copy(
                    out_s.at[b], out_hbm.at[s, :, pl.ds(bcol, 128)],
                    sem_o[b]).wait()

            def outer(o, carry):
                for b in range(2):
                    s = 2 * o + b

                    @pl.when(o >= 1)
                    def _():
                        wait_out(b, s)

                    out_b = out_s.at[b]

                    def jbody(jc, carry2):
                        j0 = jc * LANES
                        idx16 = idx_v[s, pl.ds(j0, LANES)]
                        for d0 in range(0, d_out, 8):
                            gs = [plsc.load_gather(
                                wtab,
                                [jnp.full((LANES,), d0 + k, jnp.int32), idx16])
                                for k in range(8)]
                            for k in range(8):
                                out_b[d0 + k, pl.ds(j0, LANES)] = gs[k]
                        return carry2

                    lax.fori_loop(0, 8, jbody, 0)
                    pltpu.async_copy(
                        out_s.at[b], out_hbm.at[s, :, pl.ds(bcol, 128)],
                        sem_o[b])
                return carry

            lax.fori_loop(0, seq // 2, outer, 0)
            for b in range(2):
                wait_out(b, 0)

        phase(tag_t, wtag_t, o_tag, d_tag)
        phase(rel_t, wrel_t, o_rel, d_rel)

    return small_kernel


def kernel(sent_inputs, tag_inputs, rel_inputs, W_word, W_tag, W_rel):
    bsz, seq = sent_inputs.shape
    n_vocab, d_word = W_word.shape
    d_tag = W_tag.shape[1]
    d_rel = W_rel.shape[1]
    pack = 128 // d_word
    w2 = W_word.reshape(n_vocab // pack, 128)
    small_fn = _make_small_kernel(bsz, seq, d_tag, d_rel)
    word_fn = _make_word_kernel(bsz, seq, d_word)
    o_t, o_r = small_fn(
        tag_inputs.T.astype(jnp.int32),
        rel_inputs.T.astype(jnp.int32),
        W_tag.T, W_rel.T)
    o_w = word_fn(sent_inputs.T.astype(jnp.int32), w2)
    return (
        jnp.expand_dims(jnp.transpose(o_w, (2, 0, 1)), 1),
        jnp.expand_dims(jnp.transpose(o_t, (2, 0, 1)), 1),
        jnp.expand_dims(jnp.transpose(o_r, (2, 0, 1)), 1),
    )
